# baseline (speedup 1.0000x reference)
import jax
import jax.numpy as jnp
from jax import lax
from jax.experimental import pallas as pl
from jax.experimental.pallas import tpu as pltpu
from jax.experimental.pallas import tpu_sc as plsc

ROWS = 64
COLS = 32768
LANES = 16
NWORKERS = 32
ROWS_PER_WORKER = ROWS // NWORKERS

NBLK = 16
BCHUNKS = COLS // (NBLK * LANES)
NQ = 4
QELEMS = COLS // NQ

_NEG_INF = float("-inf")
_BIG_I32 = 2**30


def _block_max(row_ref, j):
  ninf = jnp.full((LANES,), _NEG_INF, jnp.float32)
  base = j * (BCHUNKS * LANES)

  def body(c, accs):
    a0, a1, a2, a3 = accs
    o = base + c * (32 * LANES)
    for u in range(0, 32, 4):
      a0 = jnp.maximum(a0, row_ref[pl.ds(o + (u + 0) * LANES, LANES)])
      a1 = jnp.maximum(a1, row_ref[pl.ds(o + (u + 1) * LANES, LANES)])
      a2 = jnp.maximum(a2, row_ref[pl.ds(o + (u + 2) * LANES, LANES)])
      a3 = jnp.maximum(a3, row_ref[pl.ds(o + (u + 3) * LANES, LANES)])
    return a0, a1, a2, a3

  a0, a1, a2, a3 = lax.fori_loop(0, BCHUNKS // 32, body,
                                 (ninf, ninf, ninf, ninf))
  return jnp.max(jnp.maximum(jnp.maximum(a0, a1), jnp.maximum(a2, a3)))


def _rank3_blocks(bvec, lane_iota):
  ids = []
  b = bvec
  for _ in range(3):
    m = jnp.max(b)
    j = jnp.min(jnp.where(b == m, lane_iota, _BIG_I32))
    ids.append(j)
    b = jnp.where(lane_iota == j, _NEG_INF, b)
  return ids


def _argmax_blocks(row_ref, lane_iota, block_ids):
  ninf = jnp.full((LANES,), _NEG_INF, jnp.float32)
  zero = jnp.zeros((LANES,), jnp.int32)
  carry = (ninf, zero, ninf, zero, ninf, zero, ninf, zero)

  for j in block_ids:
    cbase = j * BCHUNKS

    def body(c, accs, cbase=cbase):
      b0, c0, b1, c1, b2, c2, b3, c3 = accs
      cc = cbase + c * 16
      o = cc * LANES
      bs = [b0, b1, b2, b3]
      cs = [c0, c1, c2, c3]
      for u in range(16):
        k = u % 4
        v = row_ref[pl.ds(o + u * LANES, LANES)]
        m = v > bs[k]
        bs[k] = jnp.where(m, v, bs[k])
        cs[k] = jnp.where(m, cc + u, cs[k])
      return (bs[0], cs[0], bs[1], cs[1], bs[2], cs[2], bs[3], cs[3])

    carry = lax.fori_loop(0, BCHUNKS // 16, body, carry)

  def merge(bv_a, cv_a, bv_b, cv_b):
    take = (bv_b > bv_a) | ((bv_b == bv_a) & (cv_b < cv_a))
    return jnp.where(take, bv_b, bv_a), jnp.where(take, cv_b, cv_a)

  b0, c0, b1, c1, b2, c2, b3, c3 = carry
  ba, ca = merge(b0, c0, b1, c1)
  bb, cb = merge(b2, c2, b3, c3)
  best, bestc = merge(ba, ca, bb, cb)

  idx = bestc * LANES + lane_iota
  maxv = jnp.max(best)
  return jnp.min(jnp.where(best == maxv, idx, _BIG_I32))


def _mask_out(row_ref, lane_iota, i):
  c1 = i // LANES
  l1 = i - c1 * LANES
  chunk = row_ref[pl.ds(c1 * LANES, LANES)]
  row_ref[pl.ds(c1 * LANES, LANES)] = jnp.where(
      lane_iota == l1, _NEG_INF, chunk)


def _top3_row(row_ref, lane_iota, qcopies):
  bvec = jnp.full((LANES,), _NEG_INF, jnp.float32)
  for q in range(NQ):
    qcopies[q].wait()
    for jj in range(q * (NBLK // NQ), (q + 1) * (NBLK // NQ)):
      bm = _block_max(row_ref, jj)
      bvec = jnp.where(lane_iota == jj, bm, bvec)

  j1, j2, j3 = _rank3_blocks(bvec, lane_iota)
  lo12 = jnp.minimum(j1, j2)
  hi12 = jnp.maximum(j1, j2)
  lo = jnp.minimum(lo12, j3)
  hi = jnp.maximum(hi12, j3)
  mid = j1 + j2 + j3 - lo - hi

  i1 = _argmax_blocks(row_ref, lane_iota, [j1])
  _mask_out(row_ref, lane_iota, i1)
  i2 = _argmax_blocks(row_ref, lane_iota, [lo12, hi12])
  _mask_out(row_ref, lane_iota, i2)
  i3 = _argmax_blocks(row_ref, lane_iota, [lo, mid, hi])
  return i1, i2, i3


def _sc_kernel(x_hbm, out_hbm, buf0, buf1, outbuf, *sems):
  wid = lax.axis_index("c") * 16 + lax.axis_index("s")
  r0 = wid * ROWS_PER_WORKER
  lane_iota = lax.broadcasted_iota(jnp.int32, (LANES,), 0)

  cps0 = [
      pltpu.async_copy(x_hbm.at[r0, pl.ds(q * QELEMS, QELEMS)],
                       buf0.at[pl.ds(q * QELEMS, QELEMS)], sems[q])
      for q in range(NQ)
  ]
  cps1 = [
      pltpu.async_copy(x_hbm.at[r0 + 1, pl.ds(q * QELEMS, QELEMS)],
                       buf1.at[pl.ds(q * QELEMS, QELEMS)], sems[NQ + q])
      for q in range(NQ)
  ]

  a1, a2, a3 = _top3_row(buf0, lane_iota, cps0)
  b1, b2, b3 = _top3_row(buf1, lane_iota, cps1)

  vals = [a1, a2, a3, b1, b2, b3]
  res = jnp.zeros((LANES,), jnp.float32)
  for lane, v in enumerate(vals):
    res = jnp.where(lane_iota == lane, v.astype(jnp.float32), res)
  outbuf[...] = res
  pltpu.sync_copy(outbuf, out_hbm.at[wid])


@jax.jit
def kernel(x):
  mesh = plsc.VectorSubcoreMesh(core_axis_name="c", subcore_axis_name="s")
  k = pl.kernel(
      _sc_kernel,
      out_type=jax.ShapeDtypeStruct((NWORKERS, LANES), jnp.float32),
      mesh=mesh,
      compiler_params=pltpu.CompilerParams(needs_layout_passes=False),
      scratch_types=[
          pltpu.VMEM((COLS,), jnp.float32),
          pltpu.VMEM((COLS,), jnp.float32),
          pltpu.VMEM((LANES,), jnp.float32),
      ] + [pltpu.SemaphoreType.DMA] * (2 * NQ),
  )
  staged = k(x)
  return staged[:, :6].reshape(ROWS, 3)


# speedup vs baseline: 1.1525x; 1.1525x over previous
import jax
import jax.numpy as jnp
from jax import lax
from jax.experimental import pallas as pl
from jax.experimental.pallas import tpu as pltpu
from jax.experimental.pallas import tpu_sc as plsc

ROWS = 64
COLS = 32768
LANES = 16
NWORKERS = 32
ROWS_PER_WORKER = ROWS // NWORKERS

NBLK = 16
BCHUNKS = COLS // (NBLK * LANES)
NQ = 1
QELEMS = COLS // NQ

_NEG_INF = float("-inf")
_BIG_I32 = 2**30


def _block_maxima(row_ref, lane_iota):
  ninf = jnp.full((LANES,), _NEG_INF, jnp.float32)

  def blk_body(j, bvec):
    base = j * (BCHUNKS * LANES)

    def body(c, accs):
      a0, a1, a2, a3 = accs
      o = base + c * (16 * LANES)
      for u in range(0, 16, 4):
        a0 = jnp.maximum(a0, row_ref[pl.ds(o + (u + 0) * LANES, LANES)])
        a1 = jnp.maximum(a1, row_ref[pl.ds(o + (u + 1) * LANES, LANES)])
        a2 = jnp.maximum(a2, row_ref[pl.ds(o + (u + 2) * LANES, LANES)])
        a3 = jnp.maximum(a3, row_ref[pl.ds(o + (u + 3) * LANES, LANES)])
      return a0, a1, a2, a3

    a0, a1, a2, a3 = lax.fori_loop(0, BCHUNKS // 16, body,
                                   (ninf, ninf, ninf, ninf))
    bm = jnp.max(jnp.maximum(jnp.maximum(a0, a1), jnp.maximum(a2, a3)))
    return jnp.where(lane_iota == j, bm, bvec)

  return lax.fori_loop(0, NBLK, blk_body, ninf)


def _rank3_blocks(bvec, lane_iota):
  ids = []
  b = bvec
  for _ in range(3):
    m = jnp.max(b)
    j = jnp.min(jnp.where(b == m, lane_iota, _BIG_I32))
    ids.append(j)
    b = jnp.where(lane_iota == j, _NEG_INF, b)
  return ids


def _argmax_blocks(row_ref, lane_iota, block_ids):
  ninf = jnp.full((LANES,), _NEG_INF, jnp.float32)
  zero = jnp.zeros((LANES,), jnp.int32)
  carry = (ninf, zero, ninf, zero, ninf, zero, ninf, zero)

  for j in block_ids:
    cbase = j * BCHUNKS

    def body(c, accs, cbase=cbase):
      b0, c0, b1, c1, b2, c2, b3, c3 = accs
      cc = cbase + c * 8
      o = cc * LANES
      bs = [b0, b1, b2, b3]
      cs = [c0, c1, c2, c3]
      for u in range(8):
        k = u % 4
        v = row_ref[pl.ds(o + u * LANES, LANES)]
        m = v > bs[k]
        bs[k] = jnp.where(m, v, bs[k])
        cs[k] = jnp.where(m, cc + u, cs[k])
      return (bs[0], cs[0], bs[1], cs[1], bs[2], cs[2], bs[3], cs[3])

    carry = lax.fori_loop(0, BCHUNKS // 8, body, carry)

  def merge(bv_a, cv_a, bv_b, cv_b):
    take = (bv_b > bv_a) | ((bv_b == bv_a) & (cv_b < cv_a))
    return jnp.where(take, bv_b, bv_a), jnp.where(take, cv_b, cv_a)

  b0, c0, b1, c1, b2, c2, b3, c3 = carry
  ba, ca = merge(b0, c0, b1, c1)
  bb, cb = merge(b2, c2, b3, c3)
  best, bestc = merge(ba, ca, bb, cb)

  idx = bestc * LANES + lane_iota
  maxv = jnp.max(best)
  return jnp.min(jnp.where(best == maxv, idx, _BIG_I32))


def _mask_out(row_ref, lane_iota, i):
  c1 = i // LANES
  l1 = i - c1 * LANES
  chunk = row_ref[pl.ds(c1 * LANES, LANES)]
  row_ref[pl.ds(c1 * LANES, LANES)] = jnp.where(
      lane_iota == l1, _NEG_INF, chunk)


def _top3_row(row_ref, lane_iota, qcopies):
  for cp in qcopies:
    cp.wait()
  bvec = _block_maxima(row_ref, lane_iota)

  j1, j2, j3 = _rank3_blocks(bvec, lane_iota)
  lo12 = jnp.minimum(j1, j2)
  hi12 = jnp.maximum(j1, j2)
  lo = jnp.minimum(lo12, j3)
  hi = jnp.maximum(hi12, j3)
  mid = j1 + j2 + j3 - lo - hi

  i1 = _argmax_blocks(row_ref, lane_iota, [j1])
  _mask_out(row_ref, lane_iota, i1)
  i2 = _argmax_blocks(row_ref, lane_iota, [lo12, hi12])
  _mask_out(row_ref, lane_iota, i2)
  i3 = _argmax_blocks(row_ref, lane_iota, [lo, mid, hi])
  return i1, i2, i3


def _sc_kernel(x_hbm, out_hbm, buf0, buf1, outbuf, *sems):
  wid = lax.axis_index("c") * 16 + lax.axis_index("s")
  r0 = wid * ROWS_PER_WORKER
  lane_iota = lax.broadcasted_iota(jnp.int32, (LANES,), 0)

  cps0 = [
      pltpu.async_copy(x_hbm.at[r0, pl.ds(q * QELEMS, QELEMS)],
                       buf0.at[pl.ds(q * QELEMS, QELEMS)], sems[q])
      for q in range(NQ)
  ]
  cps1 = [
      pltpu.async_copy(x_hbm.at[r0 + 1, pl.ds(q * QELEMS, QELEMS)],
                       buf1.at[pl.ds(q * QELEMS, QELEMS)], sems[NQ + q])
      for q in range(NQ)
  ]

  a1, a2, a3 = _top3_row(buf0, lane_iota, cps0)
  b1, b2, b3 = _top3_row(buf1, lane_iota, cps1)

  vals = [a1, a2, a3, b1, b2, b3]
  res = jnp.zeros((LANES,), jnp.float32)
  for lane, v in enumerate(vals):
    res = jnp.where(lane_iota == lane, v.astype(jnp.float32), res)
  outbuf[...] = res
  pltpu.sync_copy(outbuf, out_hbm.at[wid])


@jax.jit
def kernel(x):
  mesh = plsc.VectorSubcoreMesh(core_axis_name="c", subcore_axis_name="s")
  k = pl.kernel(
      _sc_kernel,
      out_type=jax.ShapeDtypeStruct((NWORKERS, LANES), jnp.float32),
      mesh=mesh,
      compiler_params=pltpu.CompilerParams(needs_layout_passes=False),
      scratch_types=[
          pltpu.VMEM((COLS,), jnp.float32),
          pltpu.VMEM((COLS,), jnp.float32),
          pltpu.VMEM((LANES,), jnp.float32),
      ] + [pltpu.SemaphoreType.DMA] * (2 * NQ),
  )
  staged = k(x)
  return staged[:, :6].reshape(ROWS, 3)


# speedup vs baseline: 1.1543x; 1.0015x over previous
import jax
import jax.numpy as jnp
from jax import lax
from jax.experimental import pallas as pl
from jax.experimental.pallas import tpu as pltpu
from jax.experimental.pallas import tpu_sc as plsc

ROWS = 64
COLS = 32768
LANES = 16
NWORKERS = 32
ROWS_PER_WORKER = ROWS // NWORKERS

NBLK = 16
BCHUNKS = COLS // (NBLK * LANES)
NQ = 2
QELEMS = COLS // NQ

_NEG_INF = float("-inf")
_BIG_I32 = 2**30


def _block_maxima(row_ref, lane_iota, j_lo, j_hi, bvec):
  ninf = jnp.full((LANES,), _NEG_INF, jnp.float32)

  def blk_body(j, bvec):
    base = j * (BCHUNKS * LANES)

    def body(c, accs):
      a0, a1, a2, a3 = accs
      o = base + c * (16 * LANES)
      for u in range(0, 16, 4):
        a0 = jnp.maximum(a0, row_ref[pl.ds(o + (u + 0) * LANES, LANES)])
        a1 = jnp.maximum(a1, row_ref[pl.ds(o + (u + 1) * LANES, LANES)])
        a2 = jnp.maximum(a2, row_ref[pl.ds(o + (u + 2) * LANES, LANES)])
        a3 = jnp.maximum(a3, row_ref[pl.ds(o + (u + 3) * LANES, LANES)])
      return a0, a1, a2, a3

    a0, a1, a2, a3 = lax.fori_loop(0, BCHUNKS // 16, body,
                                   (ninf, ninf, ninf, ninf))
    bm = jnp.max(jnp.maximum(jnp.maximum(a0, a1), jnp.maximum(a2, a3)))
    return jnp.where(lane_iota == j, bm, bvec)

  return lax.fori_loop(j_lo, j_hi, blk_body, bvec)


def _rank3_blocks(bvec, lane_iota):
  ids = []
  b = bvec
  for _ in range(3):
    m = jnp.max(b)
    j = jnp.min(jnp.where(b == m, lane_iota, _BIG_I32))
    ids.append(j)
    b = jnp.where(lane_iota == j, _NEG_INF, b)
  return ids


def _argmax_blocks(row_ref, lane_iota, block_ids):
  ninf = jnp.full((LANES,), _NEG_INF, jnp.float32)
  zero = jnp.zeros((LANES,), jnp.int32)
  carry = (ninf, zero, ninf, zero, ninf, zero, ninf, zero)

  for j in block_ids:
    cbase = j * BCHUNKS

    def body(c, accs, cbase=cbase):
      b0, c0, b1, c1, b2, c2, b3, c3 = accs
      cc = cbase + c * 8
      o = cc * LANES
      bs = [b0, b1, b2, b3]
      cs = [c0, c1, c2, c3]
      for u in range(8):
        k = u % 4
        v = row_ref[pl.ds(o + u * LANES, LANES)]
        m = v > bs[k]
        bs[k] = jnp.where(m, v, bs[k])
        cs[k] = jnp.where(m, cc + u, cs[k])
      return (bs[0], cs[0], bs[1], cs[1], bs[2], cs[2], bs[3], cs[3])

    carry = lax.fori_loop(0, BCHUNKS // 8, body, carry)

  def merge(bv_a, cv_a, bv_b, cv_b):
    take = (bv_b > bv_a) | ((bv_b == bv_a) & (cv_b < cv_a))
    return jnp.where(take, bv_b, bv_a), jnp.where(take, cv_b, cv_a)

  b0, c0, b1, c1, b2, c2, b3, c3 = carry
  ba, ca = merge(b0, c0, b1, c1)
  bb, cb = merge(b2, c2, b3, c3)
  best, bestc = merge(ba, ca, bb, cb)

  idx = bestc * LANES + lane_iota
  maxv = jnp.max(best)
  return jnp.min(jnp.where(best == maxv, idx, _BIG_I32))


def _mask_out(row_ref, lane_iota, i):
  c1 = i // LANES
  l1 = i - c1 * LANES
  chunk = row_ref[pl.ds(c1 * LANES, LANES)]
  row_ref[pl.ds(c1 * LANES, LANES)] = jnp.where(
      lane_iota == l1, _NEG_INF, chunk)


def _top3_row(row_ref, lane_iota, qcopies):
  bvec = jnp.full((LANES,), _NEG_INF, jnp.float32)
  qcopies[0].wait()
  bvec = _block_maxima(row_ref, lane_iota, 0, NBLK // 2, bvec)
  qcopies[1].wait()
  bvec = _block_maxima(row_ref, lane_iota, NBLK // 2, NBLK, bvec)

  j1, j2, j3 = _rank3_blocks(bvec, lane_iota)
  lo12 = jnp.minimum(j1, j2)
  hi12 = jnp.maximum(j1, j2)
  lo = jnp.minimum(lo12, j3)
  hi = jnp.maximum(hi12, j3)
  mid = j1 + j2 + j3 - lo - hi

  i1 = _argmax_blocks(row_ref, lane_iota, [j1])
  _mask_out(row_ref, lane_iota, i1)
  i2 = _argmax_blocks(row_ref, lane_iota, [lo12, hi12])
  _mask_out(row_ref, lane_iota, i2)
  i3 = _argmax_blocks(row_ref, lane_iota, [lo, mid, hi])
  return i1, i2, i3


def _sc_kernel(x_hbm, out_hbm, buf0, buf1, outbuf, *sems):
  wid = lax.axis_index("c") * 16 + lax.axis_index("s")
  r0 = wid * ROWS_PER_WORKER
  lane_iota = lax.broadcasted_iota(jnp.int32, (LANES,), 0)

  cps0 = [
      pltpu.async_copy(x_hbm.at[r0, pl.ds(q * QELEMS, QELEMS)],
                       buf0.at[pl.ds(q * QELEMS, QELEMS)], sems[q])
      for q in range(NQ)
  ]
  cps1 = [
      pltpu.async_copy(x_hbm.at[r0 + 1, pl.ds(q * QELEMS, QELEMS)],
                       buf1.at[pl.ds(q * QELEMS, QELEMS)], sems[NQ + q])
      for q in range(NQ)
  ]

  a1, a2, a3 = _top3_row(buf0, lane_iota, cps0)
  b1, b2, b3 = _top3_row(buf1, lane_iota, cps1)

  vals = [a1, a2, a3, b1, b2, b3]
  res = jnp.zeros((LANES,), jnp.float32)
  for lane, v in enumerate(vals):
    res = jnp.where(lane_iota == lane, v.astype(jnp.float32), res)
  outbuf[...] = res
  pltpu.sync_copy(outbuf, out_hbm.at[wid])


@jax.jit
def kernel(x):
  mesh = plsc.VectorSubcoreMesh(core_axis_name="c", subcore_axis_name="s")
  k = pl.kernel(
      _sc_kernel,
      out_type=jax.ShapeDtypeStruct((NWORKERS, LANES), jnp.float32),
      mesh=mesh,
      compiler_params=pltpu.CompilerParams(needs_layout_passes=False),
      scratch_types=[
          pltpu.VMEM((COLS,), jnp.float32),
          pltpu.VMEM((COLS,), jnp.float32),
          pltpu.VMEM((LANES,), jnp.float32),
      ] + [pltpu.SemaphoreType.DMA] * (2 * NQ),
  )
  staged = k(x)
  return staged[:, :6].reshape(ROWS, 3)
